# Initial kernel scaffold; baseline (speedup 1.0000x reference)
import jax
import jax.numpy as jnp
from jax.experimental import pallas as pl


def kernel(vis_mem, sem_mem, hh, ques_embed, hs0, cs0, W_att, W_ih, W_hh, b_lstm, Wq1, bq1, Wq2, bq2, Wa1, ba1, Wa2, ba2, Wa3, ba3, batch_ids):
    raise NotImplementedError("write your pallas kernel here")


# speedup vs baseline: 25.4036x; 25.4036x over previous
import numpy as np

import jax
import jax.numpy as jnp
from jax.experimental import pallas as pl

_TOTAL = 4096
_B = 16
_D = 300
_U = 5
_H = 100
_STEP = 2
_CHUNK = 1024
_NEG = -1e9
_BIGI = 1 << 30


def _rowmax_and_first_idx(s, lane):
    v = jnp.max(s, axis=1, keepdims=True)
    i = jnp.min(jnp.where(s == v, lane, _BIGI), axis=1, keepdims=True)
    return v, i


def _split3(x):
    hi = x.astype(jnp.bfloat16)
    r = x - hi.astype(jnp.float32)
    mid = r.astype(jnp.bfloat16)
    lo = (r - mid.astype(jnp.float32)).astype(jnp.bfloat16)
    return hi, mid, lo


def _exact_rows(onehot_bf, parts):
    f32 = jnp.float32
    hi, mid, lo = parts
    return (jnp.dot(onehot_bf, hi, preferred_element_type=f32)
            + jnp.dot(onehot_bf, mid, preferred_element_type=f32)
            + jnp.dot(onehot_bf, lo, preferred_element_type=f32))


def _fused(vis_h, vis_m, vis_l, sem_h, sem_m, sem_l, fac_h, fac_m, fac_l,
           q0, hs0, cs0, watt, wih, whh, blstm,
           wq1, bq1, wq2, bq2, wa1, ba1, wa2, ba2, wa3, ba3,
           bidr, bidc, out_ref):
    f32 = jnp.float32
    bf16 = jnp.bfloat16
    sqrt_d = np.float32(np.sqrt(_D))
    lane = jax.lax.broadcasted_iota(jnp.int32, (1, _TOTAL), 1)
    bid = bidr[...]
    row_b = jax.lax.rem(jax.lax.broadcasted_iota(jnp.int32, (_B * _U, 1), 0),
                        jnp.int32(_B))
    valid = bid == row_b

    mem_parts = ((vis_h[...], vis_m[...], vis_l[...]),
                 (sem_h[...], sem_m[...], sem_l[...]),
                 (fac_h[...], fac_m[...], fac_l[...]))
    q = q0[...]
    hs = hs0[...]
    cs = cs0[...]

    for _ in range(_STEP):
        q80 = jnp.concatenate([q] * _U, axis=0)
        qh = jnp.concatenate([hs, q80], axis=1)
        attq = jnp.dot(qh, watt[...], preferred_element_type=f32)
        attq_bf = attq.astype(bf16)
        scs = []
        for m in range(3):
            s = jax.lax.dot_general(attq_bf, mem_parts[m][0],
                                    (((1,), (1,)), ((), ())),
                                    preferred_element_type=f32) / sqrt_d
            scs.append(jnp.where(valid, s, f32(_NEG)))
        bv, bi = _rowmax_and_first_idx(scs[0], lane)
        bm = jnp.zeros_like(bi)
        for m in (1, 2):
            v, i = _rowmax_and_first_idx(scs[m], lane)
            upd = v > bv
            bv = jnp.where(upd, v, bv)
            bi = jnp.where(upd, i, bi)
            bm = jnp.where(upd, jnp.int32(m), bm)
        sv = None
        for m in range(3):
            s2 = jnp.where((bm == m) & (lane == bi), f32(-3e9), scs[m])
            v, i = _rowmax_and_first_idx(s2, lane)
            if sv is None:
                sv, si, sm = v, i, jnp.zeros_like(i)
            else:
                upd = v > sv
                sv = jnp.where(upd, v, sv)
                si = jnp.where(upd, i, si)
                sm = jnp.where(upd, jnp.int32(m), sm)
        e = jnp.exp(sv - bv)
        denom = 1.0 + e
        w1 = jnp.where(bv < f32(-0.9e9), f32(0), 1.0 / denom)
        w2 = jnp.where(sv < f32(-0.9e9), f32(0), e / denom)
        row1 = jnp.zeros((_B * _U, _D), f32)
        row2 = jnp.zeros((_B * _U, _D), f32)
        for m in range(3):
            oh1 = ((bm == m) & (lane == bi)).astype(bf16)
            oh2 = ((sm == m) & (lane == si)).astype(bf16)
            row1 = row1 + _exact_rows(oh1, mem_parts[m])
            row2 = row2 + _exact_rows(oh2, mem_parts[m])
        agg = w1 * row1 + w2 * row2
        gates = [jnp.dot(agg, wih[k], preferred_element_type=f32)
                 + jnp.dot(hs, whh[k], preferred_element_type=f32)
                 + blstm[k]
                 for k in range(4)]
        i_g = jax.nn.sigmoid(gates[0])
        f_g = jax.nn.sigmoid(gates[1])
        g_g = jnp.tanh(gates[2])
        o_g = jax.nn.sigmoid(gates[3])
        cs = f_g * cs + i_g * g_g
        hs = o_g * jnp.tanh(cs)
        hs_flat = jnp.concatenate(
            [hs[u * _B:(u + 1) * _B, :] for u in range(_U)], axis=1)
        qin = jnp.concatenate([q, hs_flat], axis=1)
        z = jnp.maximum(jnp.dot(qin, wq1[...], preferred_element_type=f32)
                        + bq1[...], f32(0))
        q = jnp.dot(z, wq2[...], preferred_element_type=f32) + bq2[...]

    q_parts = _split3(q)
    col16 = jax.lax.broadcasted_iota(jnp.int32, (1, _B), 1)
    for c in range(_TOTAL // _CHUNK):
        lo = c * _CHUNK
        fac_blk = (mem_parts[2][0][lo:lo + _CHUNK, :].astype(f32)
                   + mem_parts[2][1][lo:lo + _CHUNK, :].astype(f32)
                   + mem_parts[2][2][lo:lo + _CHUNK, :].astype(f32))
        onehot = (bidc[lo:lo + _CHUNK, :] == col16).astype(jnp.bfloat16)
        node_q = _exact_rows(onehot, q_parts)
        cat = jnp.concatenate([fac_blk, node_q], axis=1)
        h1 = jnp.maximum(jnp.dot(cat, wa1[...], preferred_element_type=f32)
                         + ba1[...], f32(0))
        h2 = jnp.maximum(jnp.dot(h1, wa2[...], preferred_element_type=f32)
                         + ba2[...], f32(0))
        out_ref[lo:lo + _CHUNK, :] = (jnp.dot(h2, wa3[...],
                                              preferred_element_type=f32)
                                      + ba3[...])


def kernel(vis_mem, sem_mem, hh, ques_embed, hs0, cs0, W_att, W_ih, W_hh,
           b_lstm, Wq1, bq1, Wq2, bq2, Wa1, ba1, Wa2, ba2, Wa3, ba3,
           batch_ids):
    f32 = jnp.float32
    bid = batch_ids.astype(jnp.int32)
    hs0u = jnp.transpose(hs0, (1, 0, 2)).reshape(_U * _B, _H)
    cs0u = jnp.transpose(cs0, (1, 0, 2)).reshape(_U * _B, _H)
    wih = W_ih.reshape(_D, 4, _H).transpose(1, 0, 2)
    whh = W_hh.reshape(_H, 4, _H).transpose(1, 0, 2)
    blstm = b_lstm.reshape(4, 1, _H)
    vparts = _split3(vis_mem.astype(f32))
    sparts = _split3(sem_mem.astype(f32))
    fparts = _split3(hh.astype(f32))

    out = pl.pallas_call(
        _fused,
        out_shape=jax.ShapeDtypeStruct((_TOTAL, 2), f32),
    )(
        *vparts, *sparts, *fparts,
        ques_embed.astype(f32), hs0u.astype(f32), cs0u.astype(f32),
        W_att.astype(f32), wih.astype(f32), whh.astype(f32),
        blstm.astype(f32),
        Wq1.astype(f32), bq1.reshape(1, 512).astype(f32),
        Wq2.astype(f32), bq2.reshape(1, _D).astype(f32),
        Wa1.astype(f32), ba1.reshape(1, 512).astype(f32),
        Wa2.astype(f32), ba2.reshape(1, 256).astype(f32),
        Wa3.astype(f32), ba3.reshape(1, 2).astype(f32),
        bid.reshape(1, _TOTAL), bid.reshape(_TOTAL, 1),
    )
    return out


# speedup vs baseline: 39.6779x; 1.5619x over previous
import numpy as np

import jax
import jax.numpy as jnp
from jax.experimental import pallas as pl

_TOTAL = 4096
_B = 16
_D = 300
_U = 5
_H = 100
_STEP = 2
_CHUNK = 512
_NEG = -1e9
_BIGI = 1 << 30


def _rowmax_and_first_idx(s, lane):
    v = jnp.max(s, axis=1, keepdims=True)
    i = jnp.min(jnp.where(s == v, lane, _BIGI), axis=1, keepdims=True)
    return v, i


def _gather_rows(onehot, mem):
    f32 = jnp.float32
    mid = (mem - mem.astype(jnp.bfloat16).astype(f32)).astype(jnp.bfloat16)
    return (jnp.dot(onehot, mem, preferred_element_type=f32)
            + jnp.dot(onehot.astype(jnp.bfloat16), mid,
                      preferred_element_type=f32))


def _exact_small_rows(onehot, x):
    f32 = jnp.float32
    hi = x.astype(jnp.bfloat16).astype(f32)
    r = x - hi
    mid = r.astype(jnp.bfloat16)
    lo = (r - mid.astype(f32)).astype(jnp.bfloat16)
    oh_bf = onehot.astype(jnp.bfloat16)
    return (jnp.dot(onehot, x, preferred_element_type=f32)
            + jnp.dot(oh_bf, mid, preferred_element_type=f32)
            + jnp.dot(oh_bf, lo, preferred_element_type=f32))


def _fused(vis, sem, fac,
           q0, hs0, cs0, watt, wih, whh, blstm,
           wq1, bq1, wq2, bq2, wa1, ba1, wa2, ba2, wa3, ba3,
           bidr, bidc, out_ref):
    f32 = jnp.float32
    sqrt_d = np.float32(np.sqrt(_D))
    lane = jax.lax.broadcasted_iota(jnp.int32, (1, _TOTAL), 1)
    bid = bidr[...]
    row_b = jax.lax.rem(jax.lax.broadcasted_iota(jnp.int32, (_B * _U, 1), 0),
                        jnp.int32(_B))
    valid = bid == row_b

    mems = (vis[...], sem[...], fac[...])
    q = q0[...]
    hs = hs0[...]
    cs = cs0[...]

    for _ in range(_STEP):
        q80 = jnp.concatenate([q] * _U, axis=0)
        qh = jnp.concatenate([hs, q80], axis=1)
        attq = jnp.dot(qh, watt[...], preferred_element_type=f32)
        scs = []
        for m in range(3):
            s = jax.lax.dot_general(attq, mems[m],
                                    (((1,), (1,)), ((), ())),
                                    preferred_element_type=f32) / sqrt_d
            scs.append(jnp.where(valid, s, f32(_NEG)))
        bv, bi = _rowmax_and_first_idx(scs[0], lane)
        bm = jnp.zeros_like(bi)
        for m in (1, 2):
            v, i = _rowmax_and_first_idx(scs[m], lane)
            upd = v > bv
            bv = jnp.where(upd, v, bv)
            bi = jnp.where(upd, i, bi)
            bm = jnp.where(upd, jnp.int32(m), bm)
        sv = None
        for m in range(3):
            s2 = jnp.where((bm == m) & (lane == bi), f32(-3e9), scs[m])
            v, i = _rowmax_and_first_idx(s2, lane)
            if sv is None:
                sv, si, sm = v, i, jnp.zeros_like(i)
            else:
                upd = v > sv
                sv = jnp.where(upd, v, sv)
                si = jnp.where(upd, i, si)
                sm = jnp.where(upd, jnp.int32(m), sm)
        e = jnp.exp(sv - bv)
        denom = 1.0 + e
        w1 = jnp.where(bv < f32(-0.9e9), f32(0), 1.0 / denom)
        w2 = jnp.where(sv < f32(-0.9e9), f32(0), e / denom)
        row1 = jnp.zeros((_B * _U, _D), f32)
        row2 = jnp.zeros((_B * _U, _D), f32)
        for m in range(3):
            oh1 = ((bm == m) & (lane == bi)).astype(f32)
            oh2 = ((sm == m) & (lane == si)).astype(f32)
            row1 = row1 + _gather_rows(oh1, mems[m])
            row2 = row2 + _gather_rows(oh2, mems[m])
        agg = w1 * row1 + w2 * row2
        gates = [jnp.dot(agg, wih[k], preferred_element_type=f32)
                 + jnp.dot(hs, whh[k], preferred_element_type=f32)
                 + blstm[k]
                 for k in range(4)]
        i_g = jax.nn.sigmoid(gates[0])
        f_g = jax.nn.sigmoid(gates[1])
        g_g = jnp.tanh(gates[2])
        o_g = jax.nn.sigmoid(gates[3])
        cs = f_g * cs + i_g * g_g
        hs = o_g * jnp.tanh(cs)
        hs_flat = jnp.concatenate(
            [hs[u * _B:(u + 1) * _B, :] for u in range(_U)], axis=1)
        qin = jnp.concatenate([q, hs_flat], axis=1)
        z = jnp.maximum(jnp.dot(qin, wq1[...], preferred_element_type=f32)
                        + bq1[...], f32(0))
        q = jnp.dot(z, wq2[...], preferred_element_type=f32) + bq2[...]

    col16 = jax.lax.broadcasted_iota(jnp.int32, (1, _B), 1)
    for c in range(_TOTAL // _CHUNK):
        lo = c * _CHUNK
        fac_blk = mems[2][lo:lo + _CHUNK, :]
        onehot = (bidc[lo:lo + _CHUNK, :] == col16).astype(f32)
        node_q = _exact_small_rows(onehot, q)
        cat = jnp.concatenate([fac_blk, node_q], axis=1)
        h1 = jnp.maximum(jnp.dot(cat, wa1[...], preferred_element_type=f32)
                         + ba1[...], f32(0))
        h2 = jnp.maximum(jnp.dot(h1, wa2[...], preferred_element_type=f32)
                         + ba2[...], f32(0))
        out_ref[lo:lo + _CHUNK, :] = (jnp.dot(h2, wa3[...],
                                              preferred_element_type=f32)
                                      + ba3[...])


def kernel(vis_mem, sem_mem, hh, ques_embed, hs0, cs0, W_att, W_ih, W_hh,
           b_lstm, Wq1, bq1, Wq2, bq2, Wa1, ba1, Wa2, ba2, Wa3, ba3,
           batch_ids):
    f32 = jnp.float32
    bid = batch_ids.astype(jnp.int32)
    hs0u = jnp.transpose(hs0, (1, 0, 2)).reshape(_U * _B, _H)
    cs0u = jnp.transpose(cs0, (1, 0, 2)).reshape(_U * _B, _H)
    wih = W_ih.reshape(_D, 4, _H).transpose(1, 0, 2)
    whh = W_hh.reshape(_H, 4, _H).transpose(1, 0, 2)
    blstm = b_lstm.reshape(4, 1, _H)

    out = pl.pallas_call(
        _fused,
        out_shape=jax.ShapeDtypeStruct((_TOTAL, 2), f32),
    )(
        vis_mem.astype(f32), sem_mem.astype(f32), hh.astype(f32),
        ques_embed.astype(f32), hs0u.astype(f32), cs0u.astype(f32),
        W_att.astype(f32), wih.astype(f32), whh.astype(f32),
        blstm.astype(f32),
        Wq1.astype(f32), bq1.reshape(1, 512).astype(f32),
        Wq2.astype(f32), bq2.reshape(1, _D).astype(f32),
        Wa1.astype(f32), ba1.reshape(1, 512).astype(f32),
        Wa2.astype(f32), ba2.reshape(1, 256).astype(f32),
        Wa3.astype(f32), ba3.reshape(1, 2).astype(f32),
        bid.reshape(1, _TOTAL), bid.reshape(_TOTAL, 1),
    )
    return out


# speedup vs baseline: 49.9490x; 1.2589x over previous
import numpy as np

import jax
import jax.numpy as jnp
from jax.experimental import pallas as pl

_TOTAL = 4096
_B = 16
_D = 300
_U = 5
_H = 100
_STEP = 2
_CHUNK = 512
_NEG = -1e9
_BIGI = 1 << 30


def _rowmax_and_first_idx(s, lane):
    v = jnp.max(s, axis=1, keepdims=True)
    i = jnp.min(jnp.where(s == v, lane, _BIGI), axis=1, keepdims=True)
    return v, i


_NT = (((1,), (1,)), ((), ()))


def _gather_rows(onehot, memt):
    f32 = jnp.float32
    mid = (memt - memt.astype(jnp.bfloat16).astype(f32)).astype(jnp.bfloat16)
    return (jax.lax.dot_general(onehot, memt, _NT,
                                preferred_element_type=f32)
            + jax.lax.dot_general(onehot.astype(jnp.bfloat16), mid, _NT,
                                  preferred_element_type=f32))


def _exact_small_rows(onehot, x):
    f32 = jnp.float32
    hi = x.astype(jnp.bfloat16).astype(f32)
    r = x - hi
    mid = r.astype(jnp.bfloat16)
    lo = (r - mid.astype(f32)).astype(jnp.bfloat16)
    oh_bf = onehot.astype(jnp.bfloat16)
    return (jnp.dot(onehot, x, preferred_element_type=f32)
            + jnp.dot(oh_bf, mid, preferred_element_type=f32)
            + jnp.dot(oh_bf, lo, preferred_element_type=f32))


def _fused(vis, sem, fac,
           q0, hs0, cs0, watt, wih, whh, blstm,
           wq1, bq1, wq2, bq2, wa1h, wa1q, ba1, wa2, ba2, wa3, ba3,
           bidr, bidc, out_ref):
    f32 = jnp.float32
    sqrt_d = np.float32(np.sqrt(_D))
    lane = jax.lax.broadcasted_iota(jnp.int32, (1, _TOTAL), 1)
    bid = bidr[...]
    row_b = jax.lax.rem(jax.lax.broadcasted_iota(jnp.int32, (_B * _U, 1), 0),
                        jnp.int32(_B))
    valid = bid == row_b

    mems = (vis[...], sem[...], fac[...])
    q = q0[...]
    hs = hs0[...]
    cs = cs0[...]

    for _ in range(_STEP):
        q80 = jnp.concatenate([q] * _U, axis=0)
        qh = jnp.concatenate([hs, q80], axis=1)
        attq = jnp.dot(qh, watt[...], preferred_element_type=f32)
        scs = []
        for m in range(3):
            s = jnp.dot(attq, mems[m], preferred_element_type=f32) / sqrt_d
            scs.append(jnp.where(valid, s, f32(_NEG)))
        bv, bi = _rowmax_and_first_idx(scs[0], lane)
        bm = jnp.zeros_like(bi)
        for m in (1, 2):
            v, i = _rowmax_and_first_idx(scs[m], lane)
            upd = v > bv
            bv = jnp.where(upd, v, bv)
            bi = jnp.where(upd, i, bi)
            bm = jnp.where(upd, jnp.int32(m), bm)
        sv = None
        for m in range(3):
            s2 = jnp.where((bm == m) & (lane == bi), f32(-3e9), scs[m])
            v, i = _rowmax_and_first_idx(s2, lane)
            if sv is None:
                sv, si, sm = v, i, jnp.zeros_like(i)
            else:
                upd = v > sv
                sv = jnp.where(upd, v, sv)
                si = jnp.where(upd, i, si)
                sm = jnp.where(upd, jnp.int32(m), sm)
        e = jnp.exp(sv - bv)
        denom = 1.0 + e
        w1 = jnp.where(bv < f32(-0.9e9), f32(0), 1.0 / denom)
        w2 = jnp.where(sv < f32(-0.9e9), f32(0), e / denom)
        row1 = jnp.zeros((_B * _U, _D), f32)
        row2 = jnp.zeros((_B * _U, _D), f32)
        for m in range(3):
            oh1 = ((bm == m) & (lane == bi)).astype(f32)
            oh2 = ((sm == m) & (lane == si)).astype(f32)
            row1 = row1 + _gather_rows(oh1, mems[m])
            row2 = row2 + _gather_rows(oh2, mems[m])
        agg = w1 * row1 + w2 * row2
        gates = [jnp.dot(agg, wih[k], preferred_element_type=f32)
                 + jnp.dot(hs, whh[k], preferred_element_type=f32)
                 + blstm[k]
                 for k in range(4)]
        i_g = jax.nn.sigmoid(gates[0])
        f_g = jax.nn.sigmoid(gates[1])
        g_g = jnp.tanh(gates[2])
        o_g = jax.nn.sigmoid(gates[3])
        cs = f_g * cs + i_g * g_g
        hs = o_g * jnp.tanh(cs)
        hs_flat = jnp.concatenate(
            [hs[u * _B:(u + 1) * _B, :] for u in range(_U)], axis=1)
        qin = jnp.concatenate([q, hs_flat], axis=1)
        z = jnp.maximum(jnp.dot(qin, wq1[...], preferred_element_type=f32)
                        + bq1[...], f32(0))
        q = jnp.dot(z, wq2[...], preferred_element_type=f32) + bq2[...]

    col16 = jax.lax.broadcasted_iota(jnp.int32, (1, _B), 1)
    for c in range(_TOTAL // _CHUNK):
        lo = c * _CHUNK
        fac_blk = mems[2][:, lo:lo + _CHUNK]
        onehot = (bidc[lo:lo + _CHUNK, :] == col16).astype(f32)
        node_q = _exact_small_rows(onehot, q)
        h1 = jnp.maximum(
            jax.lax.dot_general(fac_blk, wa1h[...], (((0,), (0,)), ((), ())),
                                preferred_element_type=f32)
            + jnp.dot(node_q, wa1q[...], preferred_element_type=f32)
            + ba1[...], f32(0))
        h2 = jnp.maximum(jnp.dot(h1, wa2[...], preferred_element_type=f32)
                         + ba2[...], f32(0))
        out_ref[lo:lo + _CHUNK, :] = (jnp.dot(h2, wa3[...],
                                              preferred_element_type=f32)
                                      + ba3[...])


def kernel(vis_mem, sem_mem, hh, ques_embed, hs0, cs0, W_att, W_ih, W_hh,
           b_lstm, Wq1, bq1, Wq2, bq2, Wa1, ba1, Wa2, ba2, Wa3, ba3,
           batch_ids):
    f32 = jnp.float32
    bid = batch_ids.astype(jnp.int32)
    hs0u = jnp.transpose(hs0, (1, 0, 2)).reshape(_U * _B, _H)
    cs0u = jnp.transpose(cs0, (1, 0, 2)).reshape(_U * _B, _H)
    wih = W_ih.reshape(_D, 4, _H).transpose(1, 0, 2)
    whh = W_hh.reshape(_H, 4, _H).transpose(1, 0, 2)
    blstm = b_lstm.reshape(4, 1, _H)
    wa1h = Wa1[:_D, :]
    wa1q = Wa1[_D:, :]

    out = pl.pallas_call(
        _fused,
        out_shape=jax.ShapeDtypeStruct((_TOTAL, 2), f32),
    )(
        vis_mem.T.astype(f32), sem_mem.T.astype(f32), hh.T.astype(f32),
        ques_embed.astype(f32), hs0u.astype(f32), cs0u.astype(f32),
        W_att.astype(f32), wih.astype(f32), whh.astype(f32),
        blstm.astype(f32),
        Wq1.astype(f32), bq1.reshape(1, 512).astype(f32),
        Wq2.astype(f32), bq2.reshape(1, _D).astype(f32),
        wa1h.astype(f32), wa1q.astype(f32), ba1.reshape(1, 512).astype(f32),
        Wa2.astype(f32), ba2.reshape(1, 256).astype(f32),
        Wa3.astype(f32), ba3.reshape(1, 2).astype(f32),
        bid.reshape(1, _TOTAL), bid.reshape(_TOTAL, 1),
    )
    return out


# speedup vs baseline: 54.5924x; 1.0930x over previous
import numpy as np

import jax
import jax.numpy as jnp
from jax.experimental import pallas as pl

_TOTAL = 4096
_B = 16
_D = 300
_U = 5
_H = 100
_STEP = 2
_CHUNK = 512
_NEG = -1e9
_BIGI = 1 << 30


def _rowmax_and_first_idx(s, lane):
    v = jnp.max(s, axis=1, keepdims=True)
    i = jnp.min(jnp.where(s == v, lane, _BIGI), axis=1, keepdims=True)
    return v, i


_NT = (((1,), (1,)), ((), ()))


def _gather_rows(onehot, memt):
    f32 = jnp.float32
    mid = (memt - memt.astype(jnp.bfloat16).astype(f32)).astype(jnp.bfloat16)
    return (jax.lax.dot_general(onehot, memt, _NT,
                                preferred_element_type=f32)
            + jax.lax.dot_general(onehot.astype(jnp.bfloat16), mid, _NT,
                                  preferred_element_type=f32))


def _exact_small_rows(onehot, x):
    f32 = jnp.float32
    hi = x.astype(jnp.bfloat16).astype(f32)
    r = x - hi
    mid = r.astype(jnp.bfloat16)
    lo = (r - mid.astype(f32)).astype(jnp.bfloat16)
    oh_bf = onehot.astype(jnp.bfloat16)
    return (jnp.dot(onehot, x, preferred_element_type=f32)
            + jnp.dot(oh_bf, mid, preferred_element_type=f32)
            + jnp.dot(oh_bf, lo, preferred_element_type=f32))


def _fused(vis, sem, fac,
           q0, hs0, cs0, watt, wih, whh, blstm,
           wq1, bq1, wq2, bq2, wa1h, wa1q, ba1, wa2, ba2, wa3, ba3,
           bidr, bidc, out_ref):
    f32 = jnp.float32
    sqrt_d = np.float32(np.sqrt(_D))
    lane = jax.lax.broadcasted_iota(jnp.int32, (1, _TOTAL), 1)
    bid = bidr[...]
    row_b = jax.lax.rem(jax.lax.broadcasted_iota(jnp.int32, (_B * _U, 1), 0),
                        jnp.int32(_B))
    valid = bid == row_b

    mems = (vis[...], sem[...], fac[...])
    q = q0[...]
    hs = hs0[...]
    cs = cs0[...]

    for _ in range(_STEP):
        q80 = jnp.concatenate([q] * _U, axis=0)
        qh = jnp.concatenate([hs, q80], axis=1)
        attq = jnp.dot(qh, watt[...], preferred_element_type=f32)
        scs = []
        for m in range(3):
            s = jnp.dot(attq, mems[m], preferred_element_type=f32) / sqrt_d
            scs.append(jnp.where(valid, s, f32(_NEG)))
        bv, bi = _rowmax_and_first_idx(scs[0], lane)
        bm = jnp.zeros_like(bi)
        for m in (1, 2):
            v, i = _rowmax_and_first_idx(scs[m], lane)
            upd = v > bv
            bv = jnp.where(upd, v, bv)
            bi = jnp.where(upd, i, bi)
            bm = jnp.where(upd, jnp.int32(m), bm)
        sv = None
        for m in range(3):
            s2 = jnp.where((bm == m) & (lane == bi), f32(-3e9), scs[m])
            v, i = _rowmax_and_first_idx(s2, lane)
            if sv is None:
                sv, si, sm = v, i, jnp.zeros_like(i)
            else:
                upd = v > sv
                sv = jnp.where(upd, v, sv)
                si = jnp.where(upd, i, si)
                sm = jnp.where(upd, jnp.int32(m), sm)
        e = jnp.exp(sv - bv)
        denom = 1.0 + e
        w1 = jnp.where(bv < f32(-0.9e9), f32(0), 1.0 / denom)
        w2 = jnp.where(sv < f32(-0.9e9), f32(0), e / denom)
        rows = jnp.zeros((2 * _B * _U, _D), f32)
        for m in range(3):
            oh = jnp.concatenate([((bm == m) & (lane == bi)),
                                  ((sm == m) & (lane == si))],
                                 axis=0).astype(f32)
            rows = rows + _gather_rows(oh, mems[m])
        agg = w1 * rows[:_B * _U, :] + w2 * rows[_B * _U:, :]
        gates = [jnp.dot(agg, wih[k], preferred_element_type=f32)
                 + jnp.dot(hs, whh[k], preferred_element_type=f32)
                 + blstm[k]
                 for k in range(4)]
        i_g = jax.nn.sigmoid(gates[0])
        f_g = jax.nn.sigmoid(gates[1])
        g_g = jnp.tanh(gates[2])
        o_g = jax.nn.sigmoid(gates[3])
        cs = f_g * cs + i_g * g_g
        hs = o_g * jnp.tanh(cs)
        hs_flat = jnp.concatenate(
            [hs[u * _B:(u + 1) * _B, :] for u in range(_U)], axis=1)
        qin = jnp.concatenate([q, hs_flat], axis=1)
        z = jnp.maximum(jnp.dot(qin, wq1[...], preferred_element_type=f32)
                        + bq1[...], f32(0))
        q = jnp.dot(z, wq2[...], preferred_element_type=f32) + bq2[...]

    col16 = jax.lax.broadcasted_iota(jnp.int32, (1, _B), 1)
    for c in range(_TOTAL // _CHUNK):
        lo = c * _CHUNK
        fac_blk = mems[2][:, lo:lo + _CHUNK]
        onehot = (bidc[lo:lo + _CHUNK, :] == col16).astype(f32)
        node_q = _exact_small_rows(onehot, q)
        h1 = jnp.maximum(
            jax.lax.dot_general(fac_blk, wa1h[...], (((0,), (0,)), ((), ())),
                                preferred_element_type=f32)
            + jnp.dot(node_q, wa1q[...], preferred_element_type=f32)
            + ba1[...], f32(0))
        h2 = jnp.maximum(jnp.dot(h1, wa2[...], preferred_element_type=f32)
                         + ba2[...], f32(0))
        out_ref[lo:lo + _CHUNK, :] = (jnp.dot(h2, wa3[...],
                                              preferred_element_type=f32)
                                      + ba3[...])


def kernel(vis_mem, sem_mem, hh, ques_embed, hs0, cs0, W_att, W_ih, W_hh,
           b_lstm, Wq1, bq1, Wq2, bq2, Wa1, ba1, Wa2, ba2, Wa3, ba3,
           batch_ids):
    f32 = jnp.float32
    bid = batch_ids.astype(jnp.int32)
    hs0u = jnp.transpose(hs0, (1, 0, 2)).reshape(_U * _B, _H)
    cs0u = jnp.transpose(cs0, (1, 0, 2)).reshape(_U * _B, _H)
    wih = W_ih.reshape(_D, 4, _H).transpose(1, 0, 2)
    whh = W_hh.reshape(_H, 4, _H).transpose(1, 0, 2)
    blstm = b_lstm.reshape(4, 1, _H)
    wa1h = Wa1[:_D, :]
    wa1q = Wa1[_D:, :]

    out = pl.pallas_call(
        _fused,
        out_shape=jax.ShapeDtypeStruct((_TOTAL, 2), f32),
    )(
        vis_mem.T.astype(f32), sem_mem.T.astype(f32), hh.T.astype(f32),
        ques_embed.astype(f32), hs0u.astype(f32), cs0u.astype(f32),
        W_att.astype(f32), wih.astype(f32), whh.astype(f32),
        blstm.astype(f32),
        Wq1.astype(f32), bq1.reshape(1, 512).astype(f32),
        Wq2.astype(f32), bq2.reshape(1, _D).astype(f32),
        wa1h.astype(f32), wa1q.astype(f32), ba1.reshape(1, 512).astype(f32),
        Wa2.astype(f32), ba2.reshape(1, 256).astype(f32),
        Wa3.astype(f32), ba3.reshape(1, 2).astype(f32),
        bid.reshape(1, _TOTAL), bid.reshape(_TOTAL, 1),
    )
    return out
